# Initial kernel scaffold; baseline (speedup 1.0000x reference)
import functools

import jax
import jax.numpy as jnp
from jax import lax
from jax.experimental import pallas as pl
from jax.experimental.pallas import tpu as pltpu
from jax.experimental.pallas import tpu_sc as plsc

N = 10000
E = 320000
D = 128
H = 512
G = 256
C = 2

NPAD = 10240
NC, NS, L = 2, 16, 16
ET = E // (NC * NS)
EB = 400
ROWS_PER_TILE = NPAD // NS
RB = 5
BLK = NPAD // RB

_MESH = plsc.VectorSubcoreMesh(core_axis_name="c", subcore_axis_name="s")


def _fill_f32(ref, n, value):
    def body(i, _):
        ref[pl.ds(i * L, L)] = jnp.full((L,), value, jnp.float32)
        return 0
    lax.fori_loop(0, n // L, body, 0)


def _zero_rows(ref, nrows):
    def body(i, _):
        r = i // 8
        c = lax.rem(i, 8)
        ref[r, pl.ds(c * L, L)] = jnp.zeros((L,), jnp.float32)
        return 0
    lax.fori_loop(0, nrows * 8, body, 0)


@functools.partial(
    pl.kernel,
    out_type=jax.ShapeDtypeStruct((NC, NPAD), jnp.float32),
    mesh=_MESH,
    scratch_types=[
        pltpu.VMEM((2000,), jnp.int32),
        pltpu.VMEM((2000,), jnp.float32),
        pltpu.VMEM((ROWS_PER_TILE,), jnp.float32),
        pltpu.VMEM_SHARED((NPAD,), jnp.float32),
    ],
)
def _deg_kernel(dst_hbm, out_hbm, idx_v, ones_v, zeros_v, acc_sh):
    core = lax.axis_index("c")
    sub = lax.axis_index("s")
    start = core * (E // NC) + sub * ET
    _fill_f32(ones_v, 2000, 1.0)
    _fill_f32(zeros_v, ROWS_PER_TILE, 0.0)
    pltpu.sync_copy(zeros_v, acc_sh.at[pl.ds(sub * ROWS_PER_TILE, ROWS_PER_TILE)])
    plsc.subcore_barrier()
    for b in range(ET // 2000):
        pltpu.sync_copy(dst_hbm.at[pl.ds(start + b * 2000, 2000)], idx_v)
        pltpu.sync_copy(ones_v, acc_sh.at[idx_v], add=True)
    plsc.subcore_barrier()
    pltpu.sync_copy(
        acc_sh.at[pl.ds(sub * ROWS_PER_TILE, ROWS_PER_TILE)],
        out_hbm.at[core, pl.ds(sub * ROWS_PER_TILE, ROWS_PER_TILE)],
    )


def _make_scatter_kernel(nchunk):
    def body(vals_hbm, src_hbm, dst_hbm, out_hbm, sidx, didx, rows, zbuf, acc_sh, sem):
        core = lax.axis_index("c")
        sub = lax.axis_index("s")
        start = core * (E // NC) + sub * ET
        _zero_rows(zbuf, 64)
        for c in range(nchunk):
            for j in range(ROWS_PER_TILE // 64):
                pltpu.sync_copy(
                    zbuf, acc_sh.at[pl.ds(sub * ROWS_PER_TILE + j * 64, 64)]
                )
            plsc.subcore_barrier()

            def step(b, _):
                base = start + b * EB
                pltpu.sync_copy(src_hbm.at[pl.ds(base, EB)], sidx)
                pltpu.sync_copy(dst_hbm.at[pl.ds(base, EB)], didx)
                pltpu.async_copy(vals_hbm.at[c].at[sidx], rows, sem).wait()
                pltpu.sync_copy(rows, acc_sh.at[didx], add=True)
                return 0

            lax.fori_loop(0, ET // EB, step, 0)
            plsc.subcore_barrier()
            pltpu.sync_copy(
                acc_sh.at[pl.ds(sub * ROWS_PER_TILE, ROWS_PER_TILE)],
                out_hbm.at[c, core, pl.ds(sub * ROWS_PER_TILE, ROWS_PER_TILE)],
            )
            if c + 1 < nchunk:
                plsc.subcore_barrier()

    return pl.kernel(
        body,
        out_type=jax.ShapeDtypeStruct((nchunk, NC, NPAD, 128), jnp.float32),
        mesh=_MESH,
        scratch_types=[
            pltpu.VMEM((EB,), jnp.int32),
            pltpu.VMEM((EB,), jnp.int32),
            pltpu.VMEM((EB, 128), jnp.float32),
            pltpu.VMEM((64, 128), jnp.float32),
            pltpu.VMEM_SHARED((NPAD, 128), jnp.float32),
            pltpu.SemaphoreType.DMA,
        ],
    )


_scatter1 = _make_scatter_kernel(1)
_scatter4 = _make_scatter_kernel(4)


def _scale_body(degp_ref, x_ref, xs_ref, inv_ref):
    deg = degp_ref[0] + degp_ref[1] + 1.0
    inv = lax.rsqrt(deg)
    xs_ref[...] = x_ref[...] * inv
    inv_ref[...] = inv


def _scale(degp, x_pad):
    return pl.pallas_call(
        _scale_body,
        grid=(RB,),
        in_specs=[
            pl.BlockSpec((NC, BLK, 1), lambda i: (0, i, 0)),
            pl.BlockSpec((BLK, D), lambda i: (i, 0)),
        ],
        out_specs=[
            pl.BlockSpec((BLK, D), lambda i: (i, 0)),
            pl.BlockSpec((BLK, 1), lambda i: (i, 0)),
        ],
        out_shape=[
            jax.ShapeDtypeStruct((NPAD, D), jnp.float32),
            jax.ShapeDtypeStruct((NPAD, 1), jnp.float32),
        ],
    )(degp, x_pad)


def _layer0_body(p_ref, xs_ref, inv_ref, w_ref, b_ref, hs_ref):
    agg = inv_ref[...] * (p_ref[0] + p_ref[1] + xs_ref[...])
    h = jnp.dot(agg, w_ref[...], preferred_element_type=jnp.float32)
    h = jnp.maximum(h + b_ref[...], 0.0)
    hs = h * inv_ref[...]
    for c in range(4):
        hs_ref[c] = hs[:, c * 128:(c + 1) * 128]


def _layer0(p0, xs, inv, W0, b0):
    return pl.pallas_call(
        _layer0_body,
        grid=(RB,),
        in_specs=[
            pl.BlockSpec((NC, BLK, D), lambda i: (0, i, 0)),
            pl.BlockSpec((BLK, D), lambda i: (i, 0)),
            pl.BlockSpec((BLK, 1), lambda i: (i, 0)),
            pl.BlockSpec((D, H), lambda i: (0, 0)),
            pl.BlockSpec((1, H), lambda i: (0, 0)),
        ],
        out_specs=pl.BlockSpec((4, BLK, 128), lambda i: (0, i, 0)),
        out_shape=jax.ShapeDtypeStruct((4, NPAD, 128), jnp.float32),
    )(p0, xs, inv, W0, b0)


def _layer1_body(p_ref, hs_ref, inv_ref, gid_ref, w_ref, b_ref, wfc_ref, bfc_ref,
                 out_ref):
    i = pl.program_id(0)
    inv = inv_ref[...]
    parts = [inv * (p_ref[c, 0] + p_ref[c, 1] + hs_ref[c]) for c in range(4)]
    agg = jnp.concatenate(parts, axis=1)
    h = jnp.dot(agg, w_ref[...], preferred_element_type=jnp.float32)
    h = jnp.maximum(h + b_ref[...], 0.0)
    q = jnp.dot(h, wfc_ref[...], preferred_element_type=jnp.float32)
    onehot_t = (lax.broadcasted_iota(jnp.int32, (G, BLK), 0)
                == gid_ref[...]).astype(jnp.float32)
    contrib = jnp.dot(onehot_t, q, preferred_element_type=jnp.float32)

    @pl.when(i == 0)
    def _():
        out_ref[...] = contrib + bfc_ref[...]

    @pl.when(i > 0)
    def _():
        out_ref[...] += contrib


def _layer1(p1, hs4, inv, gid_row, W1, b1, Wfc, bfc):
    return pl.pallas_call(
        _layer1_body,
        grid=(RB,),
        in_specs=[
            pl.BlockSpec((4, NC, BLK, 128), lambda i: (0, 0, i, 0)),
            pl.BlockSpec((4, BLK, 128), lambda i: (0, i, 0)),
            pl.BlockSpec((BLK, 1), lambda i: (i, 0)),
            pl.BlockSpec((1, BLK), lambda i: (0, i)),
            pl.BlockSpec((H, H), lambda i: (0, 0)),
            pl.BlockSpec((1, H), lambda i: (0, 0)),
            pl.BlockSpec((H, C), lambda i: (0, 0)),
            pl.BlockSpec((1, C), lambda i: (0, 0)),
        ],
        out_specs=pl.BlockSpec((G, C), lambda i: (0, 0)),
        out_shape=jax.ShapeDtypeStruct((G, C), jnp.float32),
    )(p1, hs4, inv, gid_row, W1, b1, Wfc, bfc)


def kernel(x, edge_index, node_graph_index, W0, b0, W1, b1, Wfc, bfc):
    src = edge_index[0]
    dst = edge_index[1]
    x_pad = jnp.pad(x, ((0, NPAD - N), (0, 0)))
    gid_row = jnp.pad(node_graph_index, (0, NPAD - N),
                      constant_values=G).reshape(1, NPAD)

    degp = _deg_kernel(dst).reshape(NC, NPAD, 1)
    xs, inv = _scale(degp, x_pad)
    p0 = _scatter1(xs.reshape(1, NPAD, D), src, dst)
    hs4 = _layer0(p0[0], xs, inv, W0, b0.reshape(1, H))
    p1 = _scatter4(hs4, src, dst)
    logits = _layer1(p1, hs4, inv, gid_row, W1, b1.reshape(1, H),
                     Wfc, bfc.reshape(1, C))
    return logits


# speedup vs baseline: 15.1283x; 15.1283x over previous
import functools

import jax
import jax.numpy as jnp
from jax import lax
from jax.experimental import pallas as pl
from jax.experimental.pallas import tpu as pltpu
from jax.experimental.pallas import tpu_sc as plsc

N = 10000
E = 320000
D = 128
H = 512
G = 256
C = 2

NPAD = 10240
NC, NS, L = 2, 16, 16
ET = E // (NC * NS)
EB = 200
ZR = 32
ROWS_PER_TILE = NPAD // NS
RB = 5
BLK = NPAD // RB

_MESH = plsc.VectorSubcoreMesh(core_axis_name="c", subcore_axis_name="s")


def _fill_f32(ref, n, value):
    def body(i, _):
        ref[pl.ds(i * L, L)] = jnp.full((L,), value, jnp.float32)
        return 0
    lax.fori_loop(0, n // L, body, 0)


def _zero_rows(ref, nrows):
    def body(i, _):
        r = i // 8
        c = lax.rem(i, 8)
        ref[r, pl.ds(c * L, L)] = jnp.zeros((L,), jnp.float32)
        return 0
    lax.fori_loop(0, nrows * 8, body, 0)


@functools.partial(
    pl.kernel,
    out_type=jax.ShapeDtypeStruct((NC, NPAD), jnp.float32),
    mesh=_MESH,
    scratch_types=[
        pltpu.VMEM((2000,), jnp.int32),
        pltpu.VMEM((2000,), jnp.float32),
        pltpu.VMEM((ROWS_PER_TILE,), jnp.float32),
        pltpu.VMEM_SHARED((NPAD,), jnp.float32),
    ],
)
def _deg_kernel(dst_hbm, out_hbm, idx_v, ones_v, zeros_v, acc_sh):
    core = lax.axis_index("c")
    sub = lax.axis_index("s")
    start = core * (E // NC) + sub * ET
    _fill_f32(ones_v, 2000, 1.0)
    _fill_f32(zeros_v, ROWS_PER_TILE, 0.0)
    pltpu.sync_copy(zeros_v, acc_sh.at[pl.ds(sub * ROWS_PER_TILE, ROWS_PER_TILE)])
    plsc.subcore_barrier()
    for b in range(ET // 2000):
        pltpu.sync_copy(dst_hbm.at[pl.ds(start + b * 2000, 2000)], idx_v)
        pltpu.sync_copy(ones_v, acc_sh.at[idx_v], add=True)
    plsc.subcore_barrier()
    pltpu.sync_copy(
        acc_sh.at[pl.ds(sub * ROWS_PER_TILE, ROWS_PER_TILE)],
        out_hbm.at[core, pl.ds(sub * ROWS_PER_TILE, ROWS_PER_TILE)],
    )


def _make_scatter_kernel(nchunk):
    def body(vals_hbm, src_hbm, dst_hbm, out_hbm, sidx, didx, rows, zbuf, acc_sh, sem):
        core = lax.axis_index("c")
        sub = lax.axis_index("s")
        start = core * (E // NC) + sub * ET
        _zero_rows(zbuf, ZR)
        for c in range(nchunk):
            for j in range(ROWS_PER_TILE // ZR):
                pltpu.sync_copy(
                    zbuf, acc_sh.at[pl.ds(sub * ROWS_PER_TILE + j * ZR, ZR)]
                )
            plsc.subcore_barrier()

            def step(b, _):
                base = start + b * EB
                pltpu.sync_copy(src_hbm.at[pl.ds(base, EB)], sidx)
                pltpu.sync_copy(dst_hbm.at[pl.ds(base, EB)], didx)
                pltpu.async_copy(vals_hbm.at[c].at[sidx], rows, sem).wait()
                pltpu.sync_copy(rows, acc_sh.at[didx], add=True)
                return 0

            lax.fori_loop(0, ET // EB, step, 0)
            plsc.subcore_barrier()
            pltpu.sync_copy(
                acc_sh.at[pl.ds(sub * ROWS_PER_TILE, ROWS_PER_TILE)],
                out_hbm.at[c, core, pl.ds(sub * ROWS_PER_TILE, ROWS_PER_TILE)],
            )
            if c + 1 < nchunk:
                plsc.subcore_barrier()

    return pl.kernel(
        body,
        out_type=jax.ShapeDtypeStruct((nchunk, NC, NPAD, 128), jnp.float32),
        mesh=_MESH,
        scratch_types=[
            pltpu.VMEM((EB,), jnp.int32),
            pltpu.VMEM((EB,), jnp.int32),
            pltpu.VMEM((EB, 128), jnp.float32),
            pltpu.VMEM((ZR, 128), jnp.float32),
            pltpu.VMEM_SHARED((NPAD, 128), jnp.float32),
            pltpu.SemaphoreType.DMA,
        ],
    )


_scatter1 = _make_scatter_kernel(1)
_scatter4 = _make_scatter_kernel(4)


def _scale_body(degp_ref, x_ref, xs_ref, inv_ref):
    deg = degp_ref[0] + degp_ref[1] + 1.0
    inv = lax.rsqrt(deg)
    xs_ref[...] = x_ref[...] * inv
    inv_ref[...] = inv


def _scale(degp, x_pad):
    return pl.pallas_call(
        _scale_body,
        grid=(RB,),
        in_specs=[
            pl.BlockSpec((NC, BLK, 1), lambda i: (0, i, 0)),
            pl.BlockSpec((BLK, D), lambda i: (i, 0)),
        ],
        out_specs=[
            pl.BlockSpec((BLK, D), lambda i: (i, 0)),
            pl.BlockSpec((BLK, 1), lambda i: (i, 0)),
        ],
        out_shape=[
            jax.ShapeDtypeStruct((NPAD, D), jnp.float32),
            jax.ShapeDtypeStruct((NPAD, 1), jnp.float32),
        ],
    )(degp, x_pad)


def _layer0_body(p_ref, xs_ref, inv_ref, w_ref, b_ref, hs_ref):
    agg = inv_ref[...] * (p_ref[0] + p_ref[1] + xs_ref[...])
    h = jnp.dot(agg, w_ref[...], preferred_element_type=jnp.float32)
    h = jnp.maximum(h + b_ref[...], 0.0)
    hs = h * inv_ref[...]
    for c in range(4):
        hs_ref[c] = hs[:, c * 128:(c + 1) * 128]


def _layer0(p0, xs, inv, W0, b0):
    return pl.pallas_call(
        _layer0_body,
        grid=(RB,),
        in_specs=[
            pl.BlockSpec((NC, BLK, D), lambda i: (0, i, 0)),
            pl.BlockSpec((BLK, D), lambda i: (i, 0)),
            pl.BlockSpec((BLK, 1), lambda i: (i, 0)),
            pl.BlockSpec((D, H), lambda i: (0, 0)),
            pl.BlockSpec((1, H), lambda i: (0, 0)),
        ],
        out_specs=pl.BlockSpec((4, BLK, 128), lambda i: (0, i, 0)),
        out_shape=jax.ShapeDtypeStruct((4, NPAD, 128), jnp.float32),
    )(p0, xs, inv, W0, b0)


def _layer1_body(p_ref, hs_ref, inv_ref, gid_ref, w_ref, b_ref, wfc_ref, bfc_ref,
                 out_ref):
    i = pl.program_id(0)
    inv = inv_ref[...]
    parts = [inv * (p_ref[c, 0] + p_ref[c, 1] + hs_ref[c]) for c in range(4)]
    agg = jnp.concatenate(parts, axis=1)
    h = jnp.dot(agg, w_ref[...], preferred_element_type=jnp.float32)
    h = jnp.maximum(h + b_ref[...], 0.0)
    q = jnp.dot(h, wfc_ref[...], preferred_element_type=jnp.float32)
    onehot_t = (lax.broadcasted_iota(jnp.int32, (G, BLK), 0)
                == gid_ref[...]).astype(jnp.float32)
    contrib = jnp.dot(onehot_t, q, preferred_element_type=jnp.float32)

    @pl.when(i == 0)
    def _():
        out_ref[...] = contrib + bfc_ref[...]

    @pl.when(i > 0)
    def _():
        out_ref[...] += contrib


def _layer1(p1, hs4, inv, gid_row, W1, b1, Wfc, bfc):
    return pl.pallas_call(
        _layer1_body,
        grid=(RB,),
        in_specs=[
            pl.BlockSpec((4, NC, BLK, 128), lambda i: (0, 0, i, 0)),
            pl.BlockSpec((4, BLK, 128), lambda i: (0, i, 0)),
            pl.BlockSpec((BLK, 1), lambda i: (i, 0)),
            pl.BlockSpec((1, BLK), lambda i: (0, i)),
            pl.BlockSpec((H, H), lambda i: (0, 0)),
            pl.BlockSpec((1, H), lambda i: (0, 0)),
            pl.BlockSpec((H, C), lambda i: (0, 0)),
            pl.BlockSpec((1, C), lambda i: (0, 0)),
        ],
        out_specs=pl.BlockSpec((G, C), lambda i: (0, 0)),
        out_shape=jax.ShapeDtypeStruct((G, C), jnp.float32),
    )(p1, hs4, inv, gid_row, W1, b1, Wfc, bfc)


def kernel(x, edge_index, node_graph_index, W0, b0, W1, b1, Wfc, bfc):
    src = edge_index[0]
    dst = edge_index[1]
    x_pad = jnp.pad(x, ((0, NPAD - N), (0, 0)))
    gid_row = jnp.pad(node_graph_index, (0, NPAD - N),
                      constant_values=G).reshape(1, NPAD)

    degp = _deg_kernel(dst).reshape(NC, NPAD, 1)
    xs, inv = _scale(degp, x_pad)
    p0 = _scatter1(xs.reshape(1, NPAD, D), src, dst)
    hs4 = _layer0(p0[0], xs, inv, W0, b0.reshape(1, H))
    p1 = _scatter4(hs4, src, dst)
    logits = _layer1(p1, hs4, inv, gid_row, W1, b1.reshape(1, H),
                     Wfc, bfc.reshape(1, C))
    return logits
